# Initial kernel scaffold; baseline (speedup 1.0000x reference)
import jax
import jax.numpy as jnp
from jax.experimental import pallas as pl


def kernel(x):
    raise NotImplementedError("write your pallas kernel here")


# speedup vs baseline: 2.8545x; 2.8545x over previous
import functools

import jax
import jax.numpy as jnp
from jax import lax
from jax.experimental import pallas as pl
from jax.experimental.pallas import tpu as pltpu
from jax.experimental.pallas import tpu_sc as plsc

N = 4096
K = 409
L = 16
NV = N // L


def _kwta_2d(x2d):
    rows = x2d.shape[0]
    info = plsc.get_sparse_core_info()
    nc, ns = info.num_cores, info.num_subcores
    nw = nc * ns
    assert rows % nw == 0
    rpw = rows // nw

    mesh = plsc.VectorSubcoreMesh(core_axis_name="c", subcore_axis_name="s")

    @functools.partial(
        pl.kernel,
        mesh=mesh,
        compiler_params=pltpu.CompilerParams(needs_layout_passes=False),
        out_type=jax.ShapeDtypeStruct((rows, N), jnp.float32),
        scratch_types=[
            pltpu.VMEM((N,), jnp.float32),
            pltpu.VMEM((N,), jnp.uint32),
            pltpu.VMEM((N,), jnp.float32),
        ],
    )
    def kwta(x_hbm, out_hbm, row_v, key_v, out_v):
        wid = lax.axis_index("s") * nc + lax.axis_index("c")
        row0 = wid * rpw

        def row_body(r, carry):
            row = row0 + r
            pltpu.sync_copy(x_hbm.at[row], row_v)

            def key_body(j, c):
                xv = row_v[pl.ds(j * L, L)]
                u = lax.bitcast_convert_type(xv, jnp.uint32)
                flip = jnp.where(
                    (u >> jnp.uint32(31)) != jnp.uint32(0),
                    jnp.uint32(0xFFFFFFFF),
                    jnp.uint32(0x80000000),
                )
                key_v[pl.ds(j * L, L)] = u ^ flip
                return c

            lax.fori_loop(0, NV, key_body, 0)

            def bit_body(t, cur):
                bit = lax.shift_left(jnp.uint32(1), jnp.uint32(31) - t.astype(jnp.uint32))
                cand = cur | bit

                def cnt_body(j, acc):
                    kv = key_v[pl.ds(j * L, L)]
                    return acc + jnp.where(kv >= cand, jnp.int32(1), jnp.int32(0))

                acc = lax.fori_loop(0, NV, cnt_body, jnp.zeros((L,), jnp.int32))
                cnt = jnp.sum(acc)
                return jnp.where(cnt >= K, cand, cur)

            thr = lax.fori_loop(0, 32, bit_body, jnp.uint32(0))

            def out_body(j, c):
                kv = key_v[pl.ds(j * L, L)]
                xv = row_v[pl.ds(j * L, L)]
                out_v[pl.ds(j * L, L)] = jnp.where(kv >= thr, xv, jnp.float32(0.0))
                return c

            lax.fori_loop(0, NV, out_body, 0)

            pltpu.sync_copy(out_v, out_hbm.at[row])
            return carry

        lax.fori_loop(0, rpw, row_body, 0)

    return kwta(x2d)


@jax.jit
def kernel(x):
    shp = x.shape
    out = _kwta_2d(x.reshape(-1, shp[-1]))
    return out.reshape(shp)


# speedup vs baseline: 8.8816x; 3.1114x over previous
import functools

import jax
import jax.numpy as jnp
from jax import lax
from jax.experimental import pallas as pl
from jax.experimental.pallas import tpu as pltpu
from jax.experimental.pallas import tpu_sc as plsc

N = 4096
K = 409
L = 16
NV = N // L
NB = 4096
NG = 256
PT = N - K
BIG = 1 << 30


def _kwta_2d(x2d):
    rows = x2d.shape[0]
    info = plsc.get_sparse_core_info()
    nc, ns = info.num_cores, info.num_subcores
    nw = nc * ns
    assert rows % nw == 0
    rpw = rows // nw

    mesh = plsc.VectorSubcoreMesh(core_axis_name="c", subcore_axis_name="s")

    @functools.partial(
        pl.kernel,
        mesh=mesh,
        compiler_params=pltpu.CompilerParams(needs_layout_passes=False),
        out_type=jax.ShapeDtypeStruct((rows, N), jnp.float32),
        scratch_types=[
            pltpu.VMEM((N,), jnp.float32),
            pltpu.VMEM((N,), jnp.uint32),
            pltpu.VMEM((N,), jnp.float32),
            pltpu.VMEM((NB,), jnp.int32),
            pltpu.VMEM((NG,), jnp.int32),
            pltpu.VMEM((N,), jnp.int32),
        ],
    )
    def kwta(x_hbm, out_hbm, row_v, key_v, out_v, fine_v, coarse_v, cand_v):
        wid = lax.axis_index("s") * nc + lax.axis_index("c")
        row0 = wid * rpw
        zero = jnp.zeros((L,), jnp.int32)
        ones = jnp.ones((L,), jnp.int32)
        iota = lax.iota(jnp.int32, L)

        def row_body(r, carry):
            row = row0 + r
            pltpu.sync_copy(x_hbm.at[row], row_v)

            def zf_body(j, c):
                fine_v[pl.ds(j * L, L)] = zero
                return c

            lax.fori_loop(0, NB // L, zf_body, 0)

            def zc_body(j, c):
                coarse_v[pl.ds(j * L, L)] = zero
                return c

            lax.fori_loop(0, NG // L, zc_body, 0)

            def a_body(j, c):
                xv = row_v[pl.ds(j * L, L)]
                u = lax.bitcast_convert_type(xv, jnp.uint32)
                flip = jnp.where(
                    (u >> jnp.uint32(31)) != jnp.uint32(0),
                    jnp.uint32(0xFFFFFFFF),
                    jnp.uint32(0x80000000),
                )
                key = u ^ flip
                key_v[pl.ds(j * L, L)] = key
                bi = (key >> jnp.uint32(20)).astype(jnp.int32)
                plsc.addupdate_scatter(fine_v, [bi], ones)
                plsc.addupdate_scatter(coarse_v, [bi >> 4], ones)
                return c

            lax.fori_loop(0, NV, a_body, 0)

            def c_body(j, cr):
                s, fnd, fcs, fv, fj = cr
                v = coarse_v[pl.ds(j * L, L)]
                cs = plsc.cumsum(v) + s
                mask = cs > PT
                pc = plsc.all_reduce_population_count(mask)
                any_ = jnp.where(pc > 0, 1, 0)
                new = jnp.where((any_ != 0) & (fnd == 0), True, False)
                fcs = jnp.where(new, cs, fcs)
                fv = jnp.where(new, v, fv)
                fj = jnp.where(new, jnp.full((L,), 0, jnp.int32) + j, fj)
                fnd = jnp.maximum(fnd, any_)
                s = s + jnp.sum(v)
                return (s, fnd, fcs, fv, fj)

            _, _, fcs, fv, fj = lax.fori_loop(
                0, NG // L, c_body, (jnp.int32(0), zero, zero, zero, zero)
            )
            maskf = fcs > PT
            m = jnp.sum(jnp.where(maskf, 1, 0))
            lane = L - m
            g_star = jnp.max(fj) * L + lane
            cg = jnp.min(jnp.where(maskf, fcs, BIG))
            vg = jnp.sum(jnp.where(iota == lane, fv, 0))
            p0 = cg - vg

            vf = fine_v[pl.ds(g_star * L, L)]
            csf = plsc.cumsum(vf) + p0
            mask2 = csf > PT
            m2 = jnp.sum(jnp.where(mask2, 1, 0))
            lane2 = L - m2
            cb = jnp.min(jnp.where(mask2, csf, BIG))
            r_need = jnp.int32(K) - (jnp.int32(N) - cb)
            b_star = g_star * L + lane2
            bsu = b_star.astype(jnp.uint32)

            def cp_body(j, off):
                kv = key_v[pl.ds(j * L, L)]
                msk = (kv >> jnp.uint32(20)) == bsu
                pos = plsc.cumsum(jnp.where(msk, 1, 0))
                pc = plsc.all_reduce_population_count(msk)
                dest = off + pos - 1
                low = (kv & jnp.uint32(0xFFFFF)).astype(jnp.int32)
                plsc.store_scatter(cand_v, [dest], low, mask=msk)
                return off + pc

            offv = lax.fori_loop(0, NV, cp_body, zero)
            ncand = jnp.max(offv)
            nvc = lax.div(ncand + (L - 1), jnp.int32(L))

            cur = jnp.int32(0)
            for bit in range(19, -1, -1):
                cand = cur | (1 << bit)

                def s_body(j, acc, cand=cand):
                    ck = cand_v[pl.ds(j * L, L)]
                    valid = (iota + j * L) < ncand
                    hit = valid & (ck >= cand)
                    return acc + jnp.where(hit, 1, 0)

                acc = lax.fori_loop(0, nvc, s_body, zero)
                cnt = jnp.sum(acc)
                cur = jnp.where(cnt >= r_need, cand, cur)

            thr = (bsu << jnp.uint32(20)) | cur.astype(jnp.uint32)

            def o_body(j, c):
                kv = key_v[pl.ds(j * L, L)]
                xv = row_v[pl.ds(j * L, L)]
                out_v[pl.ds(j * L, L)] = jnp.where(kv >= thr, xv, jnp.float32(0.0))
                return c

            lax.fori_loop(0, NV, o_body, 0)

            pltpu.sync_copy(out_v, out_hbm.at[row])
            return carry

        lax.fori_loop(0, rpw, row_body, 0)

    return kwta(x2d)


@jax.jit
def kernel(x):
    shp = x.shape
    out = _kwta_2d(x.reshape(-1, shp[-1]))
    return out.reshape(shp)


# speedup vs baseline: 12.0673x; 1.3587x over previous
import functools

import jax
import jax.numpy as jnp
from jax import lax
from jax.experimental import pallas as pl
from jax.experimental.pallas import tpu as pltpu
from jax.experimental.pallas import tpu_sc as plsc

N = 4096
K = 409
L = 16
NV = N // L
NB = 4096
NG = 256
PT = N - K
BIG = 1 << 30


def _kwta_2d(x2d):
    rows = x2d.shape[0]
    info = plsc.get_sparse_core_info()
    nc, ns = info.num_cores, info.num_subcores
    nw = nc * ns
    assert rows % nw == 0
    rpw = rows // nw

    mesh = plsc.VectorSubcoreMesh(core_axis_name="c", subcore_axis_name="s")

    @functools.partial(
        pl.kernel,
        mesh=mesh,
        compiler_params=pltpu.CompilerParams(needs_layout_passes=False),
        out_type=jax.ShapeDtypeStruct((rows, N), jnp.float32),
        scratch_types=[
            pltpu.VMEM((N,), jnp.float32),
            pltpu.VMEM((N,), jnp.float32),
            pltpu.VMEM((N,), jnp.uint32),
            pltpu.VMEM((N,), jnp.float32),
            pltpu.VMEM((N,), jnp.float32),
            pltpu.VMEM((NB,), jnp.int32),
            pltpu.VMEM((NG,), jnp.int32),
            pltpu.VMEM((N,), jnp.int32),
            pltpu.SemaphoreType.DMA,
            pltpu.SemaphoreType.DMA,
            pltpu.SemaphoreType.DMA,
            pltpu.SemaphoreType.DMA,
        ],
    )
    def kwta(x_hbm, out_hbm, row0_v, row1_v, key_v, out0_v, out1_v,
             fine_v, coarse_v, cand_v, isem0, isem1, osem0, osem1):
        wid = lax.axis_index("s") * nc + lax.axis_index("c")
        row0 = wid * rpw
        zero = jnp.zeros((L,), jnp.int32)
        ones = jnp.ones((L,), jnp.int32)
        iota = lax.iota(jnp.int32, L)
        isems = (isem0, isem1)
        osems = (osem0, osem1)
        rows_v = (row0_v, row1_v)
        outs_v = (out0_v, out1_v)

        pltpu.async_copy(x_hbm.at[row0], row0_v, isem0)

        def process(row, buf, next_row_valid):
            nbuf = 1 - buf
            pltpu.make_async_copy(x_hbm.at[row], rows_v[buf], isems[buf]).wait()

            if next_row_valid is True:
                pltpu.async_copy(x_hbm.at[row + 1], rows_v[nbuf], isems[nbuf])
            else:
                @pl.when(next_row_valid)
                def _():
                    pltpu.async_copy(
                        x_hbm.at[row + 1], rows_v[nbuf], isems[nbuf]
                    )

            rbuf = rows_v[buf]
            obuf = outs_v[buf]

            def zf_body(j, c):
                for u in range(8):
                    fine_v[pl.ds((j * 8 + u) * L, L)] = zero
                return c

            lax.fori_loop(0, NB // L // 8, zf_body, 0)

            def zc_body(j, c):
                for u in range(8):
                    coarse_v[pl.ds((j * 8 + u) * L, L)] = zero
                return c

            lax.fori_loop(0, NG // L // 8, zc_body, 0)

            def a_body(j, c):
                for u in range(4):
                    o = (j * 4 + u) * L
                    xv = rbuf[pl.ds(o, L)]
                    uu = lax.bitcast_convert_type(xv, jnp.uint32)
                    flip = jnp.where(
                        (uu >> jnp.uint32(31)) != jnp.uint32(0),
                        jnp.uint32(0xFFFFFFFF),
                        jnp.uint32(0x80000000),
                    )
                    key = uu ^ flip
                    key_v[pl.ds(o, L)] = key
                    bi = (key >> jnp.uint32(20)).astype(jnp.int32)
                    plsc.addupdate_scatter(fine_v, [bi], ones)
                    plsc.addupdate_scatter(coarse_v, [bi >> 4], ones)
                return c

            lax.fori_loop(0, NV // 4, a_body, 0)

            def c_body(j, cr):
                s, fnd, fcs, fv, fj = cr
                v = coarse_v[pl.ds(j * L, L)]
                cs = plsc.cumsum(v) + s
                mask = cs > PT
                pc = plsc.all_reduce_population_count(mask)
                any_ = jnp.where(pc > 0, 1, 0)
                new = jnp.where((any_ != 0) & (fnd == 0), True, False)
                fcs = jnp.where(new, cs, fcs)
                fv = jnp.where(new, v, fv)
                fj = jnp.where(new, jnp.full((L,), 0, jnp.int32) + j, fj)
                fnd = jnp.maximum(fnd, any_)
                s = s + jnp.sum(v)
                return (s, fnd, fcs, fv, fj)

            _, _, fcs, fv, fj = lax.fori_loop(
                0, NG // L, c_body, (jnp.int32(0), zero, zero, zero, zero)
            )
            maskf = fcs > PT
            m = jnp.sum(jnp.where(maskf, 1, 0))
            lane = L - m
            g_star = jnp.max(fj) * L + lane
            cg = jnp.min(jnp.where(maskf, fcs, BIG))
            vg = jnp.sum(jnp.where(iota == lane, fv, 0))
            p0 = cg - vg

            vf = fine_v[pl.ds(g_star * L, L)]
            csf = plsc.cumsum(vf) + p0
            mask2 = csf > PT
            m2 = jnp.sum(jnp.where(mask2, 1, 0))
            lane2 = L - m2
            cb = jnp.min(jnp.where(mask2, csf, BIG))
            r_need = jnp.int32(K) - (jnp.int32(N) - cb)
            b_star = g_star * L + lane2
            bsu = b_star.astype(jnp.uint32)

            def cp_body(j, off):
                for u in range(4):
                    o = (j * 4 + u) * L
                    kv = key_v[pl.ds(o, L)]
                    msk = (kv >> jnp.uint32(20)) == bsu
                    pos = plsc.cumsum(jnp.where(msk, 1, 0))
                    pc = plsc.all_reduce_population_count(msk)
                    dest = off + pos - 1
                    low = (kv & jnp.uint32(0xFFFFF)).astype(jnp.int32)
                    plsc.store_scatter(cand_v, [dest], low, mask=msk)
                    off = off + pc
                return off

            offv = lax.fori_loop(0, NV // 4, cp_body, zero)
            ncand = jnp.max(offv)
            nvc = lax.div(ncand + (L - 1), jnp.int32(L))

            cur = jnp.int32(0)
            for bit in range(19, -1, -1):
                cand = cur | (1 << bit)

                def s_body(j, acc, cand=cand):
                    ck = cand_v[pl.ds(j * L, L)]
                    valid = (iota + j * L) < ncand
                    hit = valid & (ck >= cand)
                    return acc + jnp.where(hit, 1, 0)

                acc = lax.fori_loop(0, nvc, s_body, zero)
                cnt = jnp.sum(acc)
                cur = jnp.where(cnt >= r_need, cand, cur)

            thr = (bsu << jnp.uint32(20)) | cur.astype(jnp.uint32)

            def o_body(j, c):
                for u in range(8):
                    o = (j * 8 + u) * L
                    kv = key_v[pl.ds(o, L)]
                    xv = rbuf[pl.ds(o, L)]
                    obuf[pl.ds(o, L)] = jnp.where(kv >= thr, xv, jnp.float32(0.0))
                return c

            lax.fori_loop(0, NV // 8, o_body, 0)

            pltpu.async_copy(obuf, out_hbm.at[row], osems[buf])

        npairs = rpw // 2

        def pair_body(p, carry):
            r0 = p * 2
            row = row0 + r0

            @pl.when(p > 0)
            def _():
                pltpu.make_async_copy(
                    out0_v, out_hbm.at[row - 2], osems[0]
                ).wait()

            process(row, 0, True)

            @pl.when(p > 0)
            def _():
                pltpu.make_async_copy(
                    out1_v, out_hbm.at[row - 1], osems[1]
                ).wait()

            process(row + 1, 1, p + 1 < npairs)
            return carry

        lax.fori_loop(0, npairs, pair_body, 0)

        pltpu.make_async_copy(
            out0_v, out_hbm.at[row0 + rpw - 2], osems[0]
        ).wait()
        pltpu.make_async_copy(
            out1_v, out_hbm.at[row0 + rpw - 1], osems[1]
        ).wait()

    return kwta(x2d)


@jax.jit
def kernel(x):
    shp = x.shape
    out = _kwta_2d(x.reshape(-1, shp[-1]))
    return out.reshape(shp)


# speedup vs baseline: 41.8921x; 3.4715x over previous
import functools

import jax
import jax.numpy as jnp
from jax import lax
from jax.experimental import pallas as pl
from jax.experimental.pallas import tpu as pltpu
from jax.experimental.pallas import tpu_sc as plsc

N = 4096
K = 409
L = 16
NV = N // L
NB = 4096
NG = 256
NC1 = 1024
PT = N - K
BIG = 1 << 30


def _keys_of(xv):
    u = lax.bitcast_convert_type(xv, jnp.uint32)
    flip = jnp.where(
        (u >> jnp.uint32(31)) != jnp.uint32(0),
        jnp.uint32(0xFFFFFFFF),
        jnp.uint32(0x80000000),
    )
    return u ^ flip


def _kwta_2d(x2d):
    rows = x2d.shape[0]
    info = plsc.get_sparse_core_info()
    nc, ns = info.num_cores, info.num_subcores
    nw = nc * ns
    assert rows % nw == 0
    rpw = rows // nw

    mesh = plsc.VectorSubcoreMesh(core_axis_name="c", subcore_axis_name="s")

    @functools.partial(
        pl.kernel,
        mesh=mesh,
        compiler_params=pltpu.CompilerParams(needs_layout_passes=False),
        out_type=jax.ShapeDtypeStruct((rows, N), jnp.float32),
        scratch_types=[
            pltpu.VMEM((N,), jnp.float32),
            pltpu.VMEM((N,), jnp.float32),
            pltpu.VMEM((N,), jnp.float32),
            pltpu.VMEM((N,), jnp.float32),
            pltpu.VMEM((NB,), jnp.int32),
            pltpu.VMEM((NC1,), jnp.int32),
            pltpu.VMEM((NG,), jnp.int32),
            pltpu.VMEM((N,), jnp.int32),
            pltpu.VMEM((N,), jnp.int32),
            pltpu.SemaphoreType.DMA,
            pltpu.SemaphoreType.DMA,
            pltpu.SemaphoreType.DMA,
            pltpu.SemaphoreType.DMA,
        ],
    )
    def kwta(x_hbm, out_hbm, row0_v, row1_v, out0_v, out1_v,
             fine_v, coarse_v, h2_v, cand_v, cand2_v,
             isem0, isem1, osem0, osem1):
        wid = lax.axis_index("s") * nc + lax.axis_index("c")
        row0 = wid * rpw
        zero = jnp.zeros((L,), jnp.int32)
        ones = jnp.ones((L,), jnp.int32)
        neg1 = jnp.full((L,), -1, jnp.int32)
        iota = lax.iota(jnp.int32, L)
        isems = (isem0, isem1)
        osems = (osem0, osem1)
        rows_v = (row0_v, row1_v)
        outs_v = (out0_v, out1_v)

        def scan_hist(ref, pt, nv16):
            def body(j, cr):
                s, fnd, fcs, fv, fj = cr
                v = ref[pl.ds(j * L, L)]
                cs = plsc.cumsum(v) + s
                mask = cs > pt
                pc = plsc.all_reduce_population_count(mask)
                any_ = jnp.where(pc > 0, 1, 0)
                new = jnp.where((any_ != 0) & (fnd == 0), True, False)
                fcs = jnp.where(new, cs, fcs)
                fv = jnp.where(new, v, fv)
                fj = jnp.where(new, jnp.full((L,), 0, jnp.int32) + j, fj)
                fnd = jnp.maximum(fnd, any_)
                s = s + jnp.sum(v)
                return (s, fnd, fcs, fv, fj)

            _, _, fcs, fv, fj = lax.fori_loop(
                0, nv16, body, (jnp.int32(0), zero, zero, zero, zero)
            )
            maskf = fcs > pt
            m = jnp.sum(jnp.where(maskf, 1, 0))
            lane = L - m
            b = jnp.max(fj) * L + lane
            cb = jnp.min(jnp.where(maskf, fcs, BIG))
            vb = jnp.sum(jnp.where(iota == lane, fv, 0))
            return b, cb, vb

        def process(row, buf, next_row_valid):
            nbuf = 1 - buf
            pltpu.make_async_copy(x_hbm.at[row], rows_v[buf], isems[buf]).wait()

            if next_row_valid is True:
                pltpu.async_copy(x_hbm.at[row + 1], rows_v[nbuf], isems[nbuf])
            else:
                @pl.when(next_row_valid)
                def _():
                    pltpu.async_copy(
                        x_hbm.at[row + 1], rows_v[nbuf], isems[nbuf]
                    )

            rbuf = rows_v[buf]
            obuf = outs_v[buf]

            @plsc.parallel_loop(0, NC1 // L, unroll=8)
            def _(j):
                coarse_v[pl.ds(j * L, L)] = zero

            @plsc.parallel_loop(0, NG // L, unroll=8)
            def _(j):
                h2_v[pl.ds(j * L, L)] = zero

            cand2_v[pl.ds(0, L)] = neg1

            @plsc.parallel_loop(0, NV, unroll=8)
            def _(j):
                key = _keys_of(rbuf[pl.ds(j * L, L)])
                bi = (key >> jnp.uint32(20)).astype(jnp.int32)
                plsc.addupdate_scatter(fine_v, [bi], ones)
                plsc.addupdate_scatter(coarse_v, [bi >> 2], ones)

            g1, cg, vg = scan_hist(coarse_v, PT, NC1 // L)
            p0 = cg - vg

            gv = g1 >> 2
            base = (g1 & 3) * 4
            vf = fine_v[pl.ds(gv * L, L)]
            inr = (iota >= base) & (iota < base + 4)
            csf = plsc.cumsum(jnp.where(inr, vf, 0)) + p0
            mask2 = (csf > PT) & inr
            m2 = jnp.sum(jnp.where(mask2, 1, 0))
            lane2 = base + 4 - m2
            cb = jnp.min(jnp.where(mask2, csf, BIG))
            r_need = jnp.int32(K) - (jnp.int32(N) - cb)
            b_star = gv * L + lane2
            bsu = b_star.astype(jnp.uint32)

            @plsc.parallel_loop(0, NV, unroll=8, carry=zero)
            def offv(j, off):
                kv = _keys_of(rbuf[pl.ds(j * L, L)])
                msk = (kv >> jnp.uint32(20)) == bsu
                pos = plsc.cumsum(jnp.where(msk, 1, 0))
                pc = plsc.all_reduce_population_count(msk)
                dest = off + pos - 1
                low = (kv & jnp.uint32(0xFFFFF)).astype(jnp.int32)
                plsc.store_scatter(cand_v, [dest], low, mask=msk)
                return off + pc

            ncand = jnp.max(offv)
            nvc = lax.div(ncand + (L - 1), jnp.int32(L))

            def h2_body(j, c):
                ck = cand_v[pl.ds(j * L, L)]
                valid = (iota + j * L) < ncand
                plsc.addupdate_scatter(h2_v, [ck >> 12], ones, mask=valid)
                return c

            lax.fori_loop(0, nvc, h2_body, 0)

            pt2 = ncand - r_need
            b2_star, c2b, _ = scan_hist(h2_v, pt2, NG // L)
            r2 = r_need - (ncand - c2b)
            b2u = b2_star.astype(jnp.uint32)

            def cp2_body(j, off):
                ck = cand_v[pl.ds(j * L, L)]
                valid = (iota + j * L) < ncand
                msk = valid & ((ck >> 12) == b2_star)
                pos = plsc.cumsum(jnp.where(msk, 1, 0))
                pc = plsc.all_reduce_population_count(msk)
                dest = off + pos - 1
                plsc.store_scatter(cand2_v, [dest], ck & 0xFFF, mask=msk)
                return off + pc

            off2 = lax.fori_loop(0, nvc, cp2_body, zero)
            ncand2 = jnp.max(off2)

            def via_sort(_):
                v = cand2_v[pl.ds(0, L)]
                sk, _ = plsc.sort_key_val(v, v, descending=True)
                pick = iota == (r2 - 1)
                return jnp.sum(jnp.where(pick, sk, 0))

            def via_search(_):
                nvc2 = lax.div(ncand2 + (L - 1), jnp.int32(L))
                cur = jnp.int32(0)
                for bit in range(11, -1, -1):
                    cand = cur | (1 << bit)

                    def s_body(j, acc, cand=cand):
                        ck = cand_v[pl.ds(j * L, L)]
                        valid = (iota + j * L) < ncand
                        hit = valid & ((ck >> 12) == b2_star)
                        hit = hit & ((ck & 0xFFF) >= cand)
                        return acc + jnp.where(hit, 1, 0)

                    acc = lax.fori_loop(0, nvc, s_body, zero)
                    cnt = jnp.sum(acc)
                    cur = jnp.where(cnt >= r2, cand, cur)
                del nvc2
                return cur

            low12 = lax.cond(ncand2 <= L, via_sort, via_search, 0)

            thr = (
                (bsu << jnp.uint32(20))
                | (b2u << jnp.uint32(12))
                | low12.astype(jnp.uint32)
            )
            tbits = jnp.where(
                thr >= jnp.uint32(0x80000000),
                thr ^ jnp.uint32(0x80000000),
                ~thr,
            )
            tf = lax.bitcast_convert_type(tbits, jnp.float32)

            @plsc.parallel_loop(0, NV, unroll=8)
            def _(j):
                xv = rbuf[pl.ds(j * L, L)]
                obuf[pl.ds(j * L, L)] = jnp.where(xv >= tf, xv, jnp.float32(0.0))
                fine_v[pl.ds(j * L, L)] = zero

            pltpu.async_copy(obuf, out_hbm.at[row], osems[buf])

        npairs = rpw // 2

        def pair_body(p, carry):
            r0 = p * 2
            row = row0 + r0

            @pl.when(p > 0)
            def _():
                pltpu.make_async_copy(
                    out0_v, out_hbm.at[row - 2], osems[0]
                ).wait()

            process(row, 0, True)

            @pl.when(p > 0)
            def _():
                pltpu.make_async_copy(
                    out1_v, out_hbm.at[row - 1], osems[1]
                ).wait()

            process(row + 1, 1, p + 1 < npairs)
            return carry

        pltpu.async_copy(x_hbm.at[row0], row0_v, isem0)

        @plsc.parallel_loop(0, NB // L, unroll=8)
        def _(j):
            fine_v[pl.ds(j * L, L)] = zero

        lax.fori_loop(0, npairs, pair_body, 0)

        pltpu.make_async_copy(
            out0_v, out_hbm.at[row0 + rpw - 2], osems[0]
        ).wait()
        pltpu.make_async_copy(
            out1_v, out_hbm.at[row0 + rpw - 1], osems[1]
        ).wait()

    return kwta(x2d)


@jax.jit
def kernel(x):
    shp = x.shape
    out = _kwta_2d(x.reshape(-1, shp[-1]))
    return out.reshape(shp)
